# baseline (speedup 1.0000x reference)
import functools

import jax
import jax.numpy as jnp
from jax.experimental import pallas as pl
from jax.experimental.pallas import tpu as pltpu

_VMEM_LIMIT = 64 * 1024 * 1024


def _round8(x):
    return ((x + 7) // 8) * 8


def _pick_th(H):
    for t in range(min(H, 28), 0, -1):
        if H % t == 0 and t % 2 == 0:
            return t
    return H


def _conv3_kern(x_ref, w_ref, b_ref, o_ref, s_ref, *, th, Wm, Wo, cin, cout,
                pool):
    j = pl.program_id(1)
    r0 = j * th
    pieces = []
    for dy in range(3):
        xd = x_ref[0, pl.ds(r0 + dy, th), :, :]
        for dx in range(3):
            pieces.append(xd[:, dx:dx + Wm, :])
    a = jnp.concatenate(pieces, axis=-1).reshape(th * Wm, 9 * cin)
    acc = jnp.dot(a, w_ref[...], preferred_element_type=jnp.float32)
    acc = jnp.maximum(acc + b_ref[...], 0.0)
    if pool:
        s_ref[...] = acc.reshape(th, Wm, cout // 128, 128)
        a00 = s_ref[pl.ds(0, th // 2, 2), pl.ds(0, Wo // 2, 2), :, :]
        a01 = s_ref[pl.ds(0, th // 2, 2), pl.ds(1, Wo // 2, 2), :, :]
        a10 = s_ref[pl.ds(1, th // 2, 2), pl.ds(0, Wo // 2, 2), :, :]
        a11 = s_ref[pl.ds(1, th // 2, 2), pl.ds(1, Wo // 2, 2), :, :]
        m = jnp.maximum(jnp.maximum(a00, a01), jnp.maximum(a10, a11))
        o_ref[0] = m.astype(o_ref.dtype).reshape(th // 2, Wo // 2, cout)
    else:
        o_ref[0] = acc.astype(o_ref.dtype).reshape(th, Wm, cout)[:, :Wo, :]


def _conv3x3(x, w, b, *, pool):
    N, H, W, cin = x.shape
    cout = w.shape[-1]
    Wm = _round8(W)
    Wp = _round8(Wm + 2)
    th = _pick_th(H)
    xp = jnp.pad(x, ((0, 0), (1, 1), (1, Wp - W - 1), (0, 0)))
    wr = w.astype(jnp.bfloat16).reshape(9 * cin, cout)
    br = b.astype(jnp.float32).reshape(1, cout)
    J = H // th
    if pool:
        out_shape = jax.ShapeDtypeStruct((N, H // 2, W // 2, cout), jnp.bfloat16)
        out_spec = pl.BlockSpec((1, th // 2, W // 2, cout), lambda n, j: (n, j, 0, 0))
    else:
        out_shape = jax.ShapeDtypeStruct((N, H, W, cout), jnp.bfloat16)
        out_spec = pl.BlockSpec((1, th, W, cout), lambda n, j: (n, j, 0, 0))
    return pl.pallas_call(
        functools.partial(_conv3_kern, th=th, Wm=Wm, Wo=W, cin=cin, cout=cout,
                          pool=pool),
        out_shape=out_shape,
        grid=(N, J),
        in_specs=[
            pl.BlockSpec((1, H + 2, Wp, cin), lambda n, j: (n, 0, 0, 0)),
            pl.BlockSpec((9 * cin, cout), lambda n, j: (0, 0)),
            pl.BlockSpec((1, cout), lambda n, j: (0, 0)),
        ],
        out_specs=out_spec,
        scratch_shapes=[pltpu.VMEM((th, Wm, cout // 128, 128), jnp.float32)],
        compiler_params=pltpu.CompilerParams(
            dimension_semantics=("parallel", "parallel"),
            vmem_limit_bytes=_VMEM_LIMIT),
    )(xp, wr, br)


def _conv1_kern(a_ref, w_ref, b_ref, o_ref, *, th, W, cout):
    a = a_ref[0].reshape(th * W, a_ref.shape[-1])
    acc = jnp.dot(a, w_ref[...], preferred_element_type=jnp.float32)
    acc = jnp.maximum(acc + b_ref[...], 0.0).reshape(th, W, cout)
    r1 = acc.reshape(th // 2, 2, W, cout).max(axis=1)
    r2 = r1.reshape(th // 2, W // 2, 2, cout).max(axis=2)
    o_ref[0] = r2.astype(o_ref.dtype)


def _conv1(x_nchw, w, b):
    N, cin, H, W = x_nchw.shape
    cout = w.shape[-1]
    K = 9 * cin
    Kp = _round8(K)
    xt = jnp.transpose(x_nchw, (0, 2, 3, 1)).astype(jnp.bfloat16)
    xp = jnp.pad(xt, ((0, 0), (1, 1), (1, 1), (0, 0)))
    patches = jnp.concatenate(
        [xp[:, dy:dy + H, dx:dx + W, :] for dy in range(3) for dx in range(3)]
        + [jnp.zeros((N, H, W, Kp - K), jnp.bfloat16)], axis=-1)
    wr = jnp.pad(w.astype(jnp.bfloat16).reshape(K, cout), ((0, Kp - K), (0, 0)))
    br = b.astype(jnp.float32).reshape(1, cout)
    th = _pick_th(H)
    return pl.pallas_call(
        functools.partial(_conv1_kern, th=th, W=W, cout=cout),
        out_shape=jax.ShapeDtypeStruct((N, H // 2, W // 2, cout), jnp.bfloat16),
        grid=(N, H // th),
        in_specs=[
            pl.BlockSpec((1, th, W, Kp), lambda n, j: (n, j, 0, 0)),
            pl.BlockSpec((Kp, cout), lambda n, j: (0, 0)),
            pl.BlockSpec((1, cout), lambda n, j: (0, 0)),
        ],
        out_specs=pl.BlockSpec((1, th // 2, W // 2, cout),
                               lambda n, j: (n, j, 0, 0)),
        compiler_params=pltpu.CompilerParams(
            dimension_semantics=("parallel", "parallel"),
            vmem_limit_bytes=_VMEM_LIMIT),
    )(patches, wr, br)


def _head_kern(a_ref, w_ref, b_ref, o_ref, acc_ref, *, relu):
    k = pl.program_id(1)

    @pl.when(k == 0)
    def _():
        acc_ref[...] = jnp.zeros_like(acc_ref)

    acc_ref[...] += jnp.dot(a_ref[...], w_ref[...],
                            preferred_element_type=jnp.float32)

    @pl.when(k == pl.num_programs(1) - 1)
    def _():
        r = acc_ref[...] + b_ref[...]
        if relu:
            r = jnp.maximum(r, 0.0)
        o_ref[...] = r.astype(o_ref.dtype)


def _head_mm(a, w, b, *, relu, out_dtype):
    M, K = a.shape
    N = w.shape[1]
    tk = K if K <= 4096 else 3584
    tn = N if N <= 512 else (512 if N <= 1024 else 1024)
    nk, nj = K // tk, N // tn
    return pl.pallas_call(
        functools.partial(_head_kern, relu=relu),
        out_shape=jax.ShapeDtypeStruct((M, N), out_dtype),
        grid=(nj, nk),
        in_specs=[
            pl.BlockSpec((M, tk), lambda j, k: (0, k)),
            pl.BlockSpec((tk, tn), lambda j, k: (k, j)),
            pl.BlockSpec((1, tn), lambda j, k: (0, j)),
        ],
        out_specs=pl.BlockSpec((M, tn), lambda j, k: (0, j)),
        scratch_shapes=[pltpu.VMEM((M, tn), jnp.float32)],
        compiler_params=pltpu.CompilerParams(
            dimension_semantics=("parallel", "arbitrary"),
            vmem_limit_bytes=_VMEM_LIMIT),
    )(a.astype(jnp.bfloat16), w.astype(jnp.bfloat16),
      b.astype(jnp.float32).reshape(1, N))


def kernel(x, fw0, fb0, fw1, fb1, fw2, fb2, fw3, fb3, fw4, fb4, fw5, fb5,
           fw6, fb6, fw7, fb7, hw0, hb0, hw1, hb1, hw2, hb2):
    a = _conv1(x, fw0, fb0)
    a = _conv3x3(a, fw1, fb1, pool=True)
    a = _conv3x3(a, fw2, fb2, pool=False)
    a = _conv3x3(a, fw3, fb3, pool=True)
    a = _conv3x3(a, fw4, fb4, pool=False)
    a = _conv3x3(a, fw5, fb5, pool=True)
    a = _conv3x3(a, fw6, fb6, pool=False)
    a = _conv3x3(a, fw7, fb7, pool=True)
    n = a.shape[0]
    f = a.reshape(n, -1)
    K1 = f.shape[1]
    h = _head_mm(f, hw0.reshape(K1, -1), hb0, relu=True, out_dtype=jnp.bfloat16)
    h = _head_mm(h, hw1.reshape(h.shape[1], -1), hb1, relu=True,
                 out_dtype=jnp.bfloat16)
    w3 = hw2.reshape(h.shape[1], -1)
    NC = w3.shape[1]
    NCp = ((NC + 127) // 128) * 128
    w3 = jnp.pad(w3, ((0, 0), (0, NCp - NC)))
    b3 = jnp.pad(hb2, (0, NCp - NC))
    out = _head_mm(h, w3, b3, relu=False, out_dtype=jnp.float32)
    return out[:, :NC]


# speedup vs baseline: 1.0007x; 1.0007x over previous
import functools

import jax
import jax.numpy as jnp
from jax.experimental import pallas as pl
from jax.experimental.pallas import tpu as pltpu

_VMEM_LIMIT = 64 * 1024 * 1024


def _round8(x):
    return ((x + 7) // 8) * 8


def _pick_th(H):
    for t in range(min(H, 28), 0, -1):
        if H % t == 0 and t % 2 == 0:
            return t
    return H


def _conv3_kern(x_ref, w_ref, b_ref, o_ref, *, th, Wm, Wo, cin, cout, pool):
    j = pl.program_id(1)
    r0 = j * th
    acc = None
    for dy in range(3):
        xd = x_ref[0, pl.ds(r0 + dy, th), :, :]
        a = jnp.concatenate([xd[:, dx:dx + Wm, :] for dx in range(3)],
                            axis=-1)
        a = a.reshape(th * Wm, 3 * cin)
        p = jnp.dot(a, w_ref[dy], preferred_element_type=jnp.float32)
        acc = p if acc is None else acc + p
    acc = acc + b_ref[...]
    acc = jnp.maximum(acc, 0.0).reshape(th, Wm, cout)
    if pool:
        r1 = acc.reshape(th // 2, 2, Wm, cout).max(axis=1)
        r2 = r1[:, :Wo, :].reshape(th // 2, Wo // 2, 2, cout).max(axis=2)
        o_ref[0] = r2.astype(o_ref.dtype)
    else:
        o_ref[0] = acc[:, :Wo, :].astype(o_ref.dtype)


def _conv3x3(x, w, b, *, pool):
    N, H, W, cin = x.shape
    cout = w.shape[-1]
    Wm = _round8(W)
    Wp = _round8(Wm + 2)
    th = _pick_th(H)
    xp = jnp.pad(x, ((0, 0), (1, 1), (1, Wp - W - 1), (0, 0)))
    wr = w.astype(jnp.bfloat16).reshape(3, 3 * cin, cout)
    br = b.astype(jnp.float32).reshape(1, cout)
    J = H // th
    if pool:
        out_shape = jax.ShapeDtypeStruct((N, H // 2, W // 2, cout), jnp.bfloat16)
        out_spec = pl.BlockSpec((1, th // 2, W // 2, cout), lambda n, j: (n, j, 0, 0))
    else:
        out_shape = jax.ShapeDtypeStruct((N, H, W, cout), jnp.bfloat16)
        out_spec = pl.BlockSpec((1, th, W, cout), lambda n, j: (n, j, 0, 0))
    return pl.pallas_call(
        functools.partial(_conv3_kern, th=th, Wm=Wm, Wo=W, cin=cin, cout=cout,
                          pool=pool),
        out_shape=out_shape,
        grid=(N, J),
        in_specs=[
            pl.BlockSpec((1, H + 2, Wp, cin), lambda n, j: (n, 0, 0, 0)),
            pl.BlockSpec((3, 3 * cin, cout), lambda n, j: (0, 0, 0)),
            pl.BlockSpec((1, cout), lambda n, j: (0, 0)),
        ],
        out_specs=out_spec,
        compiler_params=pltpu.CompilerParams(
            dimension_semantics=("parallel", "parallel"),
            vmem_limit_bytes=_VMEM_LIMIT),
    )(xp, wr, br)


def _conv1_kern(a_ref, w_ref, b_ref, o_ref, *, th, W, cout):
    a = a_ref[0].reshape(th * W, a_ref.shape[-1])
    acc = jnp.dot(a, w_ref[...], preferred_element_type=jnp.float32)
    acc = jnp.maximum(acc + b_ref[...], 0.0).reshape(th, W, cout)
    r1 = acc.reshape(th // 2, 2, W, cout).max(axis=1)
    r2 = r1.reshape(th // 2, W // 2, 2, cout).max(axis=2)
    o_ref[0] = r2.astype(o_ref.dtype)


def _conv1(x_nchw, w, b):
    N, cin, H, W = x_nchw.shape
    cout = w.shape[-1]
    K = 9 * cin
    Kp = _round8(K)
    xt = jnp.transpose(x_nchw, (0, 2, 3, 1)).astype(jnp.bfloat16)
    xp = jnp.pad(xt, ((0, 0), (1, 1), (1, 1), (0, 0)))
    patches = jnp.concatenate(
        [xp[:, dy:dy + H, dx:dx + W, :] for dy in range(3) for dx in range(3)]
        + [jnp.zeros((N, H, W, Kp - K), jnp.bfloat16)], axis=-1)
    wr = jnp.pad(w.astype(jnp.bfloat16).reshape(K, cout), ((0, Kp - K), (0, 0)))
    br = b.astype(jnp.float32).reshape(1, cout)
    th = _pick_th(H)
    return pl.pallas_call(
        functools.partial(_conv1_kern, th=th, W=W, cout=cout),
        out_shape=jax.ShapeDtypeStruct((N, H // 2, W // 2, cout), jnp.bfloat16),
        grid=(N, H // th),
        in_specs=[
            pl.BlockSpec((1, th, W, Kp), lambda n, j: (n, j, 0, 0)),
            pl.BlockSpec((Kp, cout), lambda n, j: (0, 0)),
            pl.BlockSpec((1, cout), lambda n, j: (0, 0)),
        ],
        out_specs=pl.BlockSpec((1, th // 2, W // 2, cout),
                               lambda n, j: (n, j, 0, 0)),
        compiler_params=pltpu.CompilerParams(
            dimension_semantics=("parallel", "parallel"),
            vmem_limit_bytes=_VMEM_LIMIT),
    )(patches, wr, br)


def _head_kern(a_ref, w_ref, b_ref, o_ref, acc_ref, *, relu):
    k = pl.program_id(1)

    @pl.when(k == 0)
    def _():
        acc_ref[...] = jnp.zeros_like(acc_ref)

    acc_ref[...] += jnp.dot(a_ref[...], w_ref[...],
                            preferred_element_type=jnp.float32)

    @pl.when(k == pl.num_programs(1) - 1)
    def _():
        r = acc_ref[...] + b_ref[...]
        if relu:
            r = jnp.maximum(r, 0.0)
        o_ref[...] = r.astype(o_ref.dtype)


def _head_mm(a, w, b, *, relu, out_dtype):
    M, K = a.shape
    N = w.shape[1]
    tk = K if K <= 4096 else 3584
    tn = N if N <= 512 else (512 if N <= 1024 else 1024)
    nk, nj = K // tk, N // tn
    return pl.pallas_call(
        functools.partial(_head_kern, relu=relu),
        out_shape=jax.ShapeDtypeStruct((M, N), out_dtype),
        grid=(nj, nk),
        in_specs=[
            pl.BlockSpec((M, tk), lambda j, k: (0, k)),
            pl.BlockSpec((tk, tn), lambda j, k: (k, j)),
            pl.BlockSpec((1, tn), lambda j, k: (0, j)),
        ],
        out_specs=pl.BlockSpec((M, tn), lambda j, k: (0, j)),
        scratch_shapes=[pltpu.VMEM((M, tn), jnp.float32)],
        compiler_params=pltpu.CompilerParams(
            dimension_semantics=("parallel", "arbitrary"),
            vmem_limit_bytes=_VMEM_LIMIT),
    )(a.astype(jnp.bfloat16), w.astype(jnp.bfloat16),
      b.astype(jnp.float32).reshape(1, N))


def kernel(x, fw0, fb0, fw1, fb1, fw2, fb2, fw3, fb3, fw4, fb4, fw5, fb5,
           fw6, fb6, fw7, fb7, hw0, hb0, hw1, hb1, hw2, hb2):
    a = _conv1(x, fw0, fb0)
    a = _conv3x3(a, fw1, fb1, pool=True)
    a = _conv3x3(a, fw2, fb2, pool=False)
    a = _conv3x3(a, fw3, fb3, pool=True)
    a = _conv3x3(a, fw4, fb4, pool=False)
    a = _conv3x3(a, fw5, fb5, pool=True)
    a = _conv3x3(a, fw6, fb6, pool=False)
    a = _conv3x3(a, fw7, fb7, pool=True)
    n = a.shape[0]
    f = a.reshape(n, -1)
    K1 = f.shape[1]
    h = _head_mm(f, hw0.reshape(K1, -1), hb0, relu=True, out_dtype=jnp.bfloat16)
    h = _head_mm(h, hw1.reshape(h.shape[1], -1), hb1, relu=True,
                 out_dtype=jnp.bfloat16)
    w3 = hw2.reshape(h.shape[1], -1)
    NC = w3.shape[1]
    NCp = ((NC + 127) // 128) * 128
    w3 = jnp.pad(w3, ((0, 0), (0, NCp - NC)))
    b3 = jnp.pad(hb2, (0, NCp - NC))
    out = _head_mm(h, w3, b3, relu=False, out_dtype=jnp.float32)
    return out[:, :NC]


# speedup vs baseline: 2.7120x; 2.7100x over previous
import functools

import jax
import jax.numpy as jnp
from jax.experimental import pallas as pl
from jax.experimental.pallas import tpu as pltpu

_VMEM_LIMIT = 64 * 1024 * 1024


def _round8(x):
    return ((x + 7) // 8) * 8


def _pick_th(H):
    for t in range(min(H, 28), 0, -1):
        if H % t == 0 and t % 2 == 0:
            return t
    return H


def _conv3_kern(x_ref, w_ref, b_ref, o_ref, *, th, Wm, Wo, cin, cout, pool):
    j = pl.program_id(1)
    r0 = j * th
    acc = None
    for dy in range(3):
        xd = x_ref[0, pl.ds(r0 + dy, th), :, :]
        a = jnp.concatenate([xd[:, dx:dx + Wm, :] for dx in range(3)],
                            axis=-1)
        a = a.reshape(th * Wm, 3 * cin)
        p = jnp.dot(a, w_ref[dy], preferred_element_type=jnp.float32)
        acc = p if acc is None else acc + p
    acc = acc + b_ref[...]
    acc = jnp.maximum(acc, 0.0).reshape(th, Wm, cout)
    if pool:
        r1 = acc.reshape(th // 2, 2, Wm, cout).max(axis=1)
        r2 = r1[:, :Wo, :].reshape(th // 2, Wo // 2, 2, cout).max(axis=2)
        o_ref[0] = r2.astype(o_ref.dtype)
    else:
        o_ref[0] = acc[:, :Wo, :].astype(o_ref.dtype)


def _conv3x3(x, w, b, *, pool):
    N, H, W, cin = x.shape
    cout = w.shape[-1]
    Wm = _round8(W)
    Wp = _round8(Wm + 2)
    th = _pick_th(H)
    xp = jnp.pad(x, ((0, 0), (1, 1), (1, Wp - W - 1), (0, 0)))
    wr = w.astype(jnp.bfloat16).reshape(3, 3 * cin, cout)
    br = b.astype(jnp.float32).reshape(1, cout)
    J = H // th
    if pool:
        out_shape = jax.ShapeDtypeStruct((N, H // 2, W // 2, cout), jnp.bfloat16)
        out_spec = pl.BlockSpec((1, th // 2, W // 2, cout), lambda n, j: (n, j, 0, 0))
    else:
        out_shape = jax.ShapeDtypeStruct((N, H, W, cout), jnp.bfloat16)
        out_spec = pl.BlockSpec((1, th, W, cout), lambda n, j: (n, j, 0, 0))
    return pl.pallas_call(
        functools.partial(_conv3_kern, th=th, Wm=Wm, Wo=W, cin=cin, cout=cout,
                          pool=pool),
        out_shape=out_shape,
        grid=(N, J),
        in_specs=[
            pl.BlockSpec((1, H + 2, Wp, cin), lambda n, j: (n, 0, 0, 0)),
            pl.BlockSpec((3, 3 * cin, cout), lambda n, j: (0, 0, 0)),
            pl.BlockSpec((1, cout), lambda n, j: (0, 0)),
        ],
        out_specs=out_spec,
        compiler_params=pltpu.CompilerParams(
            dimension_semantics=("parallel", "parallel"),
            vmem_limit_bytes=_VMEM_LIMIT),
    )(xp, wr, br)


def _conv1_kern(a_ref, w_ref, b_ref, o_ref, *, th, W, cout):
    a = a_ref[0].reshape(th * W, a_ref.shape[-1])
    acc = jnp.dot(a, w_ref[...], preferred_element_type=jnp.float32)
    acc = jnp.maximum(acc + b_ref[...], 0.0).reshape(th, W, cout)
    r1 = acc.reshape(th // 2, 2, W, cout).max(axis=1)
    r2 = r1.reshape(th // 2, W // 2, 2, cout).max(axis=2)
    o_ref[0] = r2.astype(o_ref.dtype)


def _conv1(x_nchw, w, b):
    N, cin, H, W = x_nchw.shape
    cout = w.shape[-1]
    K = 9 * cin
    Kp = _round8(K)
    xt = jnp.transpose(x_nchw, (0, 2, 3, 1)).astype(jnp.bfloat16)
    xp = jnp.pad(xt, ((0, 0), (1, 1), (1, 1), (0, 0)))
    patches = jnp.concatenate(
        [xp[:, dy:dy + H, dx:dx + W, :] for dy in range(3) for dx in range(3)],
        axis=-1)
    patches = jnp.pad(patches, ((0, 0), (0, 0), (0, 0), (0, Kp - K)))
    wr = jnp.pad(w.astype(jnp.bfloat16).reshape(K, cout), ((0, Kp - K), (0, 0)))
    br = b.astype(jnp.float32).reshape(1, cout)
    th = _pick_th(H)
    return pl.pallas_call(
        functools.partial(_conv1_kern, th=th, W=W, cout=cout),
        out_shape=jax.ShapeDtypeStruct((N, H // 2, W // 2, cout), jnp.bfloat16),
        grid=(N, H // th),
        in_specs=[
            pl.BlockSpec((1, th, W, Kp), lambda n, j: (n, j, 0, 0)),
            pl.BlockSpec((Kp, cout), lambda n, j: (0, 0)),
            pl.BlockSpec((1, cout), lambda n, j: (0, 0)),
        ],
        out_specs=pl.BlockSpec((1, th // 2, W // 2, cout),
                               lambda n, j: (n, j, 0, 0)),
        compiler_params=pltpu.CompilerParams(
            dimension_semantics=("parallel", "parallel"),
            vmem_limit_bytes=_VMEM_LIMIT),
    )(patches, wr, br)


def _head_kern(a_ref, w_ref, b_ref, o_ref, acc_ref, *, relu):
    k = pl.program_id(1)

    @pl.when(k == 0)
    def _():
        acc_ref[...] = jnp.zeros_like(acc_ref)

    acc_ref[...] += jnp.dot(a_ref[...], w_ref[...],
                            preferred_element_type=jnp.float32)

    @pl.when(k == pl.num_programs(1) - 1)
    def _():
        r = acc_ref[...] + b_ref[...]
        if relu:
            r = jnp.maximum(r, 0.0)
        o_ref[...] = r.astype(o_ref.dtype)


def _head_mm(a, w, b, *, relu, out_dtype):
    M, K = a.shape
    N = w.shape[1]
    tk = K if K <= 4096 else 3584
    tn = N if N <= 512 else (512 if N <= 1024 else 1024)
    nk, nj = K // tk, N // tn
    return pl.pallas_call(
        functools.partial(_head_kern, relu=relu),
        out_shape=jax.ShapeDtypeStruct((M, N), out_dtype),
        grid=(nj, nk),
        in_specs=[
            pl.BlockSpec((M, tk), lambda j, k: (0, k)),
            pl.BlockSpec((tk, tn), lambda j, k: (k, j)),
            pl.BlockSpec((1, tn), lambda j, k: (0, j)),
        ],
        out_specs=pl.BlockSpec((M, tn), lambda j, k: (0, j)),
        scratch_shapes=[pltpu.VMEM((M, tn), jnp.float32)],
        compiler_params=pltpu.CompilerParams(
            dimension_semantics=("parallel", "arbitrary"),
            vmem_limit_bytes=_VMEM_LIMIT),
    )(a.astype(jnp.bfloat16), w.astype(jnp.bfloat16),
      b.astype(jnp.float32).reshape(1, N))


def kernel(x, fw0, fb0, fw1, fb1, fw2, fb2, fw3, fb3, fw4, fb4, fw5, fb5,
           fw6, fb6, fw7, fb7, hw0, hb0, hw1, hb1, hw2, hb2):
    a = _conv1(x, fw0, fb0)
    a = _conv3x3(a, fw1, fb1, pool=True)
    a = _conv3x3(a, fw2, fb2, pool=False)
    a = _conv3x3(a, fw3, fb3, pool=True)
    a = _conv3x3(a, fw4, fb4, pool=False)
    a = _conv3x3(a, fw5, fb5, pool=True)
    a = _conv3x3(a, fw6, fb6, pool=False)
    a = _conv3x3(a, fw7, fb7, pool=True)
    n = a.shape[0]
    f = a.reshape(n, -1)
    K1 = f.shape[1]
    h = _head_mm(f, hw0.reshape(K1, -1), hb0, relu=True, out_dtype=jnp.bfloat16)
    h = _head_mm(h, hw1.reshape(h.shape[1], -1), hb1, relu=True,
                 out_dtype=jnp.bfloat16)
    w3 = hw2.reshape(h.shape[1], -1)
    NC = w3.shape[1]
    NCp = ((NC + 127) // 128) * 128
    w3 = jnp.pad(w3, ((0, 0), (0, NCp - NC)))
    b3 = jnp.pad(hb2, (0, NCp - NC))
    out = _head_mm(h, w3, b3, relu=False, out_dtype=jnp.float32)
    return out[:, :NC]


# speedup vs baseline: 2.7223x; 1.0038x over previous
import functools

import jax
import jax.numpy as jnp
from jax.experimental import pallas as pl
from jax.experimental.pallas import tpu as pltpu

_VMEM_LIMIT = 64 * 1024 * 1024


def _round8(x):
    return ((x + 7) // 8) * 8


def _pick_th(H):
    for t in range(min(H, 28), 0, -1):
        if H % t == 0 and t % 2 == 0:
            return t
    return H


def _conv3_kern(x_ref, w_ref, b_ref, o_ref, *, th, Wm, Wo, cin, cout, pool):
    j = pl.program_id(1)
    r0 = j * th
    acc = None
    for dy in range(3):
        xd = x_ref[0, pl.ds(r0 + dy, th), :, :]
        a = jnp.concatenate([xd[:, dx:dx + Wm, :] for dx in range(3)],
                            axis=-1)
        a = a.reshape(th * Wm, 3 * cin)
        p = jnp.dot(a, w_ref[dy], preferred_element_type=jnp.float32)
        acc = p if acc is None else acc + p
    acc = acc + b_ref[...]
    acc = jnp.maximum(acc, 0.0).reshape(th, Wm, cout)
    if pool:
        r1 = acc.reshape(th // 2, 2, Wm, cout).max(axis=1)
        r2 = r1[:, :Wo, :].reshape(th // 2, Wo // 2, 2, cout).max(axis=2)
        o_ref[0] = r2.astype(o_ref.dtype)
    else:
        o_ref[0] = acc[:, :Wo, :].astype(o_ref.dtype)


def _conv3x3(x, w, b, *, pool):
    N, H, W, cin = x.shape
    cout = w.shape[-1]
    Wm = _round8(W)
    Wp = _round8(Wm + 2)
    th = _pick_th(H)
    xp = jnp.pad(x, ((0, 0), (1, 1), (1, Wp - W - 1), (0, 0)))
    wr = w.astype(jnp.bfloat16).reshape(3, 3 * cin, cout)
    br = b.astype(jnp.float32).reshape(1, cout)
    J = H // th
    if pool:
        out_shape = jax.ShapeDtypeStruct((N, H // 2, W // 2, cout), jnp.bfloat16)
        out_spec = pl.BlockSpec((1, th // 2, W // 2, cout), lambda n, j: (n, j, 0, 0))
    else:
        out_shape = jax.ShapeDtypeStruct((N, H, W, cout), jnp.bfloat16)
        out_spec = pl.BlockSpec((1, th, W, cout), lambda n, j: (n, j, 0, 0))
    return pl.pallas_call(
        functools.partial(_conv3_kern, th=th, Wm=Wm, Wo=W, cin=cin, cout=cout,
                          pool=pool),
        out_shape=out_shape,
        grid=(N, J),
        in_specs=[
            pl.BlockSpec((1, H + 2, Wp, cin), lambda n, j: (n, 0, 0, 0)),
            pl.BlockSpec((3, 3 * cin, cout), lambda n, j: (0, 0, 0)),
            pl.BlockSpec((1, cout), lambda n, j: (0, 0)),
        ],
        out_specs=out_spec,
        compiler_params=pltpu.CompilerParams(
            dimension_semantics=("parallel", "parallel"),
            vmem_limit_bytes=_VMEM_LIMIT),
    )(xp, wr, br)


def _conv1_kern(a_ref, w_ref, b_ref, o_ref, *, th, W, cout):
    a = a_ref[0].reshape(th * W, a_ref.shape[-1])
    acc = jnp.dot(a, w_ref[...], preferred_element_type=jnp.float32)
    acc = jnp.maximum(acc + b_ref[...], 0.0).reshape(th, W, cout)
    r1 = acc.reshape(th // 2, 2, W, cout).max(axis=1)
    r2 = r1.reshape(th // 2, W // 2, 2, cout).max(axis=2)
    o_ref[0] = r2.astype(o_ref.dtype)


def _conv1(x_nchw, w, b):
    N, cin, H, W = x_nchw.shape
    cout = w.shape[-1]
    K = 9 * cin
    Kp = _round8(K)
    xt = jnp.transpose(x_nchw, (0, 2, 3, 1)).astype(jnp.bfloat16)
    xp = jnp.pad(xt, ((0, 0), (1, 1), (1, 1), (0, 0)))
    patches = jnp.concatenate(
        [xp[:, dy:dy + H, dx:dx + W, :] for dy in range(3) for dx in range(3)],
        axis=-1)
    patches = jnp.pad(patches, ((0, 0), (0, 0), (0, 0), (0, Kp - K)))
    wr = jnp.pad(w.astype(jnp.bfloat16).reshape(K, cout), ((0, Kp - K), (0, 0)))
    br = b.astype(jnp.float32).reshape(1, cout)
    th = 56 if H % 56 == 0 else _pick_th(H)
    return pl.pallas_call(
        functools.partial(_conv1_kern, th=th, W=W, cout=cout),
        out_shape=jax.ShapeDtypeStruct((N, H // 2, W // 2, cout), jnp.bfloat16),
        grid=(N, H // th),
        in_specs=[
            pl.BlockSpec((1, th, W, Kp), lambda n, j: (n, j, 0, 0)),
            pl.BlockSpec((Kp, cout), lambda n, j: (0, 0)),
            pl.BlockSpec((1, cout), lambda n, j: (0, 0)),
        ],
        out_specs=pl.BlockSpec((1, th // 2, W // 2, cout),
                               lambda n, j: (n, j, 0, 0)),
        compiler_params=pltpu.CompilerParams(
            dimension_semantics=("parallel", "parallel"),
            vmem_limit_bytes=_VMEM_LIMIT),
    )(patches, wr, br)


def _head_kern(a_ref, w_ref, b_ref, o_ref, acc_ref, *, relu):
    k = pl.program_id(1)

    @pl.when(k == 0)
    def _():
        acc_ref[...] = jnp.zeros_like(acc_ref)

    acc_ref[...] += jnp.dot(a_ref[...], w_ref[...],
                            preferred_element_type=jnp.float32)

    @pl.when(k == pl.num_programs(1) - 1)
    def _():
        r = acc_ref[...] + b_ref[...]
        if relu:
            r = jnp.maximum(r, 0.0)
        o_ref[...] = r.astype(o_ref.dtype)


def _head_mm(a, w, b, *, relu, out_dtype):
    M, K = a.shape
    N = w.shape[1]
    tk = K if K <= 4096 else 3584
    tn = N if N <= 512 else (512 if N <= 1024 else 1024)
    nk, nj = K // tk, N // tn
    return pl.pallas_call(
        functools.partial(_head_kern, relu=relu),
        out_shape=jax.ShapeDtypeStruct((M, N), out_dtype),
        grid=(nj, nk),
        in_specs=[
            pl.BlockSpec((M, tk), lambda j, k: (0, k)),
            pl.BlockSpec((tk, tn), lambda j, k: (k, j)),
            pl.BlockSpec((1, tn), lambda j, k: (0, j)),
        ],
        out_specs=pl.BlockSpec((M, tn), lambda j, k: (0, j)),
        scratch_shapes=[pltpu.VMEM((M, tn), jnp.float32)],
        compiler_params=pltpu.CompilerParams(
            dimension_semantics=("parallel", "arbitrary"),
            vmem_limit_bytes=_VMEM_LIMIT),
    )(a.astype(jnp.bfloat16), w.astype(jnp.bfloat16),
      b.astype(jnp.float32).reshape(1, N))


def kernel(x, fw0, fb0, fw1, fb1, fw2, fb2, fw3, fb3, fw4, fb4, fw5, fb5,
           fw6, fb6, fw7, fb7, hw0, hb0, hw1, hb1, hw2, hb2):
    a = _conv1(x, fw0, fb0)
    a = _conv3x3(a, fw1, fb1, pool=True)
    a = _conv3x3(a, fw2, fb2, pool=False)
    a = _conv3x3(a, fw3, fb3, pool=True)
    a = _conv3x3(a, fw4, fb4, pool=False)
    a = _conv3x3(a, fw5, fb5, pool=True)
    a = _conv3x3(a, fw6, fb6, pool=False)
    a = _conv3x3(a, fw7, fb7, pool=True)
    n = a.shape[0]
    f = a.reshape(n, -1)
    K1 = f.shape[1]
    h = _head_mm(f, hw0.reshape(K1, -1), hb0, relu=True, out_dtype=jnp.bfloat16)
    h = _head_mm(h, hw1.reshape(h.shape[1], -1), hb1, relu=True,
                 out_dtype=jnp.bfloat16)
    w3 = hw2.reshape(h.shape[1], -1)
    NC = w3.shape[1]
    NCp = ((NC + 127) // 128) * 128
    w3 = jnp.pad(w3, ((0, 0), (0, NCp - NC)))
    b3 = jnp.pad(hb2, (0, NCp - NC))
    out = _head_mm(h, w3, b3, relu=False, out_dtype=jnp.float32)
    return out[:, :NC]


# speedup vs baseline: 2.7421x; 1.0073x over previous
import functools

import jax
import jax.numpy as jnp
from jax.experimental import pallas as pl
from jax.experimental.pallas import tpu as pltpu

_VMEM_LIMIT = 64 * 1024 * 1024


def _round8(x):
    return ((x + 7) // 8) * 8


def _pick_th(H):
    for t in range(min(H, 28), 0, -1):
        if H % t == 0 and t % 2 == 0:
            return t
    return H


def _conv3_kern(x_ref, w_ref, b_ref, o_ref, *, th, Wm, Wo, cin, cout, pool):
    j = pl.program_id(1)
    r0 = j * th
    acc = None
    for dy in range(3):
        xd = x_ref[0, pl.ds(r0 + dy, th), :, :]
        a = jnp.concatenate([xd[:, dx:dx + Wm, :] for dx in range(3)],
                            axis=-1)
        a = a.reshape(th * Wm, 3 * cin)
        p = jnp.dot(a, w_ref[dy], preferred_element_type=jnp.float32)
        acc = p if acc is None else acc + p
    acc = acc + b_ref[...]
    acc = jnp.maximum(acc, 0.0).reshape(th, Wm, cout)
    if pool:
        r1 = acc.reshape(th // 2, 2, Wm, cout).max(axis=1)
        r2 = r1[:, :Wo, :].reshape(th // 2, Wo // 2, 2, cout).max(axis=2)
        o_ref[0] = r2.astype(o_ref.dtype)
    else:
        o_ref[0] = acc[:, :Wo, :].astype(o_ref.dtype)


def _conv3x3(x, w, b, *, pool):
    N, H, W, cin = x.shape
    cout = w.shape[-1]
    Wm = _round8(W)
    Wp = _round8(Wm + 2)
    th = 56 if (H % 56 == 0 and W >= 56) else _pick_th(H)
    xp = jnp.pad(x, ((0, 0), (1, 1), (1, Wp - W - 1), (0, 0)))
    wr = w.astype(jnp.bfloat16).reshape(3, 3 * cin, cout)
    br = b.astype(jnp.float32).reshape(1, cout)
    J = H // th
    if pool:
        out_shape = jax.ShapeDtypeStruct((N, H // 2, W // 2, cout), jnp.bfloat16)
        out_spec = pl.BlockSpec((1, th // 2, W // 2, cout), lambda n, j: (n, j, 0, 0))
    else:
        out_shape = jax.ShapeDtypeStruct((N, H, W, cout), jnp.bfloat16)
        out_spec = pl.BlockSpec((1, th, W, cout), lambda n, j: (n, j, 0, 0))
    return pl.pallas_call(
        functools.partial(_conv3_kern, th=th, Wm=Wm, Wo=W, cin=cin, cout=cout,
                          pool=pool),
        out_shape=out_shape,
        grid=(N, J),
        in_specs=[
            pl.BlockSpec((1, H + 2, Wp, cin), lambda n, j: (n, 0, 0, 0)),
            pl.BlockSpec((3, 3 * cin, cout), lambda n, j: (0, 0, 0)),
            pl.BlockSpec((1, cout), lambda n, j: (0, 0)),
        ],
        out_specs=out_spec,
        compiler_params=pltpu.CompilerParams(
            dimension_semantics=("parallel", "parallel"),
            vmem_limit_bytes=_VMEM_LIMIT),
    )(xp, wr, br)


def _conv1_kern(a_ref, w_ref, b_ref, o_ref, *, th, W, cout):
    a = a_ref[0].reshape(th * W, a_ref.shape[-1])
    acc = jnp.dot(a, w_ref[...], preferred_element_type=jnp.float32)
    acc = jnp.maximum(acc + b_ref[...], 0.0).reshape(th, W, cout)
    r1 = acc.reshape(th // 2, 2, W, cout).max(axis=1)
    r2 = r1.reshape(th // 2, W // 2, 2, cout).max(axis=2)
    o_ref[0] = r2.astype(o_ref.dtype)


def _conv1(x_nchw, w, b):
    N, cin, H, W = x_nchw.shape
    cout = w.shape[-1]
    K = 9 * cin
    Kp = _round8(K)
    xt = jnp.transpose(x_nchw, (0, 2, 3, 1)).astype(jnp.bfloat16)
    xp = jnp.pad(xt, ((0, 0), (1, 1), (1, 1), (0, 0)))
    patches = jnp.concatenate(
        [xp[:, dy:dy + H, dx:dx + W, :] for dy in range(3) for dx in range(3)],
        axis=-1)
    patches = jnp.pad(patches, ((0, 0), (0, 0), (0, 0), (0, Kp - K)))
    wr = jnp.pad(w.astype(jnp.bfloat16).reshape(K, cout), ((0, Kp - K), (0, 0)))
    br = b.astype(jnp.float32).reshape(1, cout)
    th = 56 if H % 56 == 0 else _pick_th(H)
    return pl.pallas_call(
        functools.partial(_conv1_kern, th=th, W=W, cout=cout),
        out_shape=jax.ShapeDtypeStruct((N, H // 2, W // 2, cout), jnp.bfloat16),
        grid=(N, H // th),
        in_specs=[
            pl.BlockSpec((1, th, W, Kp), lambda n, j: (n, j, 0, 0)),
            pl.BlockSpec((Kp, cout), lambda n, j: (0, 0)),
            pl.BlockSpec((1, cout), lambda n, j: (0, 0)),
        ],
        out_specs=pl.BlockSpec((1, th // 2, W // 2, cout),
                               lambda n, j: (n, j, 0, 0)),
        compiler_params=pltpu.CompilerParams(
            dimension_semantics=("parallel", "parallel"),
            vmem_limit_bytes=_VMEM_LIMIT),
    )(patches, wr, br)


def _head_kern(a_ref, w_ref, b_ref, o_ref, acc_ref, *, relu):
    k = pl.program_id(1)

    @pl.when(k == 0)
    def _():
        acc_ref[...] = jnp.zeros_like(acc_ref)

    acc_ref[...] += jnp.dot(a_ref[...], w_ref[...],
                            preferred_element_type=jnp.float32)

    @pl.when(k == pl.num_programs(1) - 1)
    def _():
        r = acc_ref[...] + b_ref[...]
        if relu:
            r = jnp.maximum(r, 0.0)
        o_ref[...] = r.astype(o_ref.dtype)


def _head_mm(a, w, b, *, relu, out_dtype):
    M, K = a.shape
    N = w.shape[1]
    tk = K if K <= 4096 else 3584
    tn = N if N <= 512 else (512 if N <= 1024 else 1024)
    nk, nj = K // tk, N // tn
    return pl.pallas_call(
        functools.partial(_head_kern, relu=relu),
        out_shape=jax.ShapeDtypeStruct((M, N), out_dtype),
        grid=(nj, nk),
        in_specs=[
            pl.BlockSpec((M, tk), lambda j, k: (0, k)),
            pl.BlockSpec((tk, tn), lambda j, k: (k, j)),
            pl.BlockSpec((1, tn), lambda j, k: (0, j)),
        ],
        out_specs=pl.BlockSpec((M, tn), lambda j, k: (0, j)),
        scratch_shapes=[pltpu.VMEM((M, tn), jnp.float32)],
        compiler_params=pltpu.CompilerParams(
            dimension_semantics=("parallel", "arbitrary"),
            vmem_limit_bytes=_VMEM_LIMIT),
    )(a.astype(jnp.bfloat16), w.astype(jnp.bfloat16),
      b.astype(jnp.float32).reshape(1, N))


def kernel(x, fw0, fb0, fw1, fb1, fw2, fb2, fw3, fb3, fw4, fb4, fw5, fb5,
           fw6, fb6, fw7, fb7, hw0, hb0, hw1, hb1, hw2, hb2):
    a = _conv1(x, fw0, fb0)
    a = _conv3x3(a, fw1, fb1, pool=True)
    a = _conv3x3(a, fw2, fb2, pool=False)
    a = _conv3x3(a, fw3, fb3, pool=True)
    a = _conv3x3(a, fw4, fb4, pool=False)
    a = _conv3x3(a, fw5, fb5, pool=True)
    a = _conv3x3(a, fw6, fb6, pool=False)
    a = _conv3x3(a, fw7, fb7, pool=True)
    n = a.shape[0]
    f = a.reshape(n, -1)
    K1 = f.shape[1]
    h = _head_mm(f, hw0.reshape(K1, -1), hb0, relu=True, out_dtype=jnp.bfloat16)
    h = _head_mm(h, hw1.reshape(h.shape[1], -1), hb1, relu=True,
                 out_dtype=jnp.bfloat16)
    w3 = hw2.reshape(h.shape[1], -1)
    NC = w3.shape[1]
    NCp = ((NC + 127) // 128) * 128
    w3 = jnp.pad(w3, ((0, 0), (0, NCp - NC)))
    b3 = jnp.pad(hb2, (0, NCp - NC))
    out = _head_mm(h, w3, b3, relu=False, out_dtype=jnp.float32)
    return out[:, :NC]
